# Initial kernel scaffold; baseline (speedup 1.0000x reference)
import jax
import jax.numpy as jnp
from jax.experimental import pallas as pl


def kernel(h, edge_index, W1, b1, W2, b2):
    raise NotImplementedError("write your pallas kernel here")


# speedup vs baseline: 11.9639x; 11.9639x over previous
import functools

import jax
import jax.numpy as jnp
from jax import lax
from jax.experimental import pallas as pl
from jax.experimental.pallas import tpu as pltpu
from jax.experimental.pallas import tpu_sc as plsc

N = 10000
E = 320000
D = 128

NPAD = 10240
EPAD = 327680
CHUNK = 128
NW = 32
CPW = EPAD // (NW * CHUNK)
EPW = CPW * CHUNK
ROWS_PER_TILE = NPAD // 16


def _sc_agg(h_pad, edges3d):
    mesh = plsc.VectorSubcoreMesh(core_axis_name="c", subcore_axis_name="s")

    @functools.partial(
        pl.kernel,
        mesh=mesh,
        out_type=[
            jax.ShapeDtypeStruct((2, NPAD, D), jnp.float32),
            jax.ShapeDtypeStruct((2, NPAD), jnp.float32),
        ],
        scratch_types=[
            pltpu.VMEM((2, 2, CHUNK), jnp.int32),
            pltpu.VMEM((2, CHUNK, D), jnp.float32),
            pltpu.VMEM((CHUNK,), jnp.float32),
            pltpu.VMEM_SHARED((NPAD, D), jnp.float32),
            pltpu.VMEM_SHARED((NPAD,), jnp.float32),
            pltpu.SemaphoreType.DMA,
            pltpu.SemaphoreType.DMA,
        ],
    )
    def agg(h_hbm, e_hbm, sums_out, cnt_out,
            idx, rows, ones_v, acc, acc_cnt, sem_i, sem_g):
        c = lax.axis_index("c")
        s = lax.axis_index("s")
        w = c * 16 + s
        g0 = w * CPW

        zeros16 = jnp.zeros((16,), jnp.float32)
        ones16 = jnp.ones((16,), jnp.float32)

        def zrow(r, carry):
            for kk in range(8):
                rows[0, r, pl.ds(kk * 16, 16)] = zeros16
            return carry
        lax.fori_loop(0, CHUNK, zrow, 0)
        for kk in range(CHUNK // 16):
            ones_v[pl.ds(kk * 16, 16)] = ones16

        slab = s * ROWS_PER_TILE
        for t in range(ROWS_PER_TILE // CHUNK):
            pltpu.sync_copy(rows.at[0], acc.at[pl.ds(slab + t * CHUNK, CHUNK)])
            pltpu.sync_copy(rows.at[0, 0],
                            acc_cnt.at[pl.ds(slab + t * CHUNK, CHUNK)])

        plsc.subcore_barrier()

        pltpu.sync_copy(e_hbm.at[g0], idx.at[0])
        pltpu.async_copy(h_hbm.at[idx.at[0, 0]], rows.at[0], sem_g)
        pltpu.async_copy(e_hbm.at[g0 + 1], idx.at[1], sem_i)

        def chunk_pair(jj, carry):
            for b in range(2):
                j = jj * 2 + b

                @pl.when(j + 1 < CPW)
                def _():
                    pltpu.make_async_copy(
                        e_hbm.at[g0], idx.at[1 - b], sem_i).wait()
                    pltpu.async_copy(
                        h_hbm.at[idx.at[1 - b, 0]], rows.at[1 - b], sem_g)

                pltpu.make_async_copy(
                    h_hbm.at[idx.at[b, 0]], rows.at[b], sem_g).wait()
                pltpu.sync_copy(rows.at[b], acc.at[idx.at[b, 1]], add=True)
                pltpu.sync_copy(ones_v, acc_cnt.at[idx.at[b, 1]], add=True)

                @pl.when(j + 2 < CPW)
                def _():
                    pltpu.async_copy(e_hbm.at[g0 + j + 2], idx.at[b], sem_i)
            return carry
        lax.fori_loop(0, CPW // 2, chunk_pair, 0)

        plsc.subcore_barrier()

        for t in range(ROWS_PER_TILE // CHUNK):
            pltpu.sync_copy(
                acc.at[pl.ds(slab + t * CHUNK, CHUNK)],
                sums_out.at[c, pl.ds(slab + t * CHUNK, CHUNK)])
        pltpu.sync_copy(acc_cnt.at[pl.ds(slab, ROWS_PER_TILE)],
                        cnt_out.at[c, pl.ds(slab, ROWS_PER_TILE)])

    return agg(h_pad, edges3d)


def _tc_layer(h_pad, sums_p, cnt_t, wa, wb, b2d):
    blk = 1024

    def body(h_ref, s_ref, c_ref, wa_ref, wb_ref, b_ref, o_ref):
        sums = s_ref[0] + s_ref[1]
        cnt = jnp.sum(c_ref[...], axis=1, keepdims=True)
        mean = sums * (1.0 / jnp.maximum(cnt, 1.0))
        acc = jnp.dot(h_ref[...], wa_ref[...], preferred_element_type=jnp.float32)
        acc = acc + jnp.dot(mean, wb_ref[...], preferred_element_type=jnp.float32)
        o_ref[...] = jnp.maximum(acc + b_ref[...], 0.0)

    return pl.pallas_call(
        body,
        grid=(NPAD // blk,),
        in_specs=[
            pl.BlockSpec((blk, D), lambda i: (i, 0)),
            pl.BlockSpec((2, blk, D), lambda i: (0, i, 0)),
            pl.BlockSpec((blk, 2), lambda i: (i, 0)),
            pl.BlockSpec((D, D), lambda i: (0, 0)),
            pl.BlockSpec((D, D), lambda i: (0, 0)),
            pl.BlockSpec((1, D), lambda i: (0, 0)),
        ],
        out_specs=pl.BlockSpec((blk, D), lambda i: (i, 0)),
        out_shape=jax.ShapeDtypeStruct((NPAD, D), jnp.float32),
    )(h_pad, sums_p, cnt_t, wa, wb, b2d)


def kernel(h, edge_index, W1, b1, W2, b2):
    src = edge_index[0].astype(jnp.int32)
    dst = edge_index[1].astype(jnp.int32)

    epad = EPAD - E
    pad_ar = jnp.arange(epad, dtype=jnp.int32)
    src_p = jnp.concatenate([src, (pad_ar * 37) % NPAD])
    dst_p = jnp.concatenate([dst, N + pad_ar % (NPAD - N)])
    edges3d = jnp.stack(
        [src_p.reshape(EPAD // CHUNK, CHUNK),
         dst_p.reshape(EPAD // CHUNK, CHUNK)], axis=1)
    h_p = jnp.pad(h, ((0, NPAD - N), (0, 0)))

    sums_p, cnt_all = _sc_agg(h_p, edges3d)
    cnt_t = cnt_all.T
    h_p = _tc_layer(h_p, sums_p, cnt_t, W1[:D], W1[D:], b1.reshape(1, D))
    sums_p2, _ = _sc_agg(h_p, edges3d)
    h_p = _tc_layer(h_p, sums_p2, cnt_t, W2[:D], W2[D:], b2.reshape(1, D))
    return h_p[:N]
